# Initial kernel scaffold; baseline (speedup 1.0000x reference)
import jax
import jax.numpy as jnp
from jax.experimental import pallas as pl


def kernel(features, edge_index, edge_weight, W):
    raise NotImplementedError("write your pallas kernel here")


# speedup vs baseline: 3.5217x; 3.5217x over previous
import functools

import jax
import jax.numpy as jnp
from jax import lax
from jax.experimental import pallas as pl
from jax.experimental.pallas import tpu as pltpu
from jax.experimental.pallas import tpu_sc as plsc

NC = 2
NS = 16
NW = NC * NS
LANES = 16
SUB = 128
PHASES = 2


def _make_sc_spmm(NP, DIN, EP):
    EPW = EP // NW
    NSUB = EPW // SUB
    PNS = NSUB // PHASES
    PPAIR = PNS // 2
    RPT = NP // NS

    mesh = plsc.VectorSubcoreMesh(core_axis_name="c", subcore_axis_name="s")

    @functools.partial(
        pl.kernel,
        out_type=jax.ShapeDtypeStruct((NC, NP, DIN), jnp.float32),
        mesh=mesh,
        scratch_types=[
            pltpu.VMEM_SHARED((NP, DIN), jnp.float32),
            pltpu.VMEM((PNS, SUB), jnp.int32),
            pltpu.VMEM((PNS, SUB), jnp.int32),
            pltpu.VMEM((PNS, SUB), jnp.float32),
            pltpu.VMEM((SUB, DIN), jnp.float32),
            pltpu.VMEM((SUB, DIN), jnp.float32),
            pltpu.SemaphoreType.DMA,
            pltpu.SemaphoreType.DMA,
        ],
    )
    def sc_spmm(feat_hbm, src_hbm, dst_hbm, w_hbm, out_hbm,
                acc, src_v, dst_v, w_v, buf0, buf1, sem0, sem1):
        c = lax.axis_index("c")
        s = lax.axis_index("s")
        wid = s * NC + c

        zero16 = jnp.zeros((LANES,), jnp.float32)

        def zbody(i, carry):
            for d in range(DIN // LANES):
                buf0[i, pl.ds(d * LANES, LANES)] = zero16
            return carry

        lax.fori_loop(0, SUB, zbody, 0)
        for z in range(RPT // SUB):
            pltpu.sync_copy(buf0, acc.at[pl.ds(s * RPT + z * SUB, SUB)])
        rem = RPT % SUB
        if rem:
            pltpu.sync_copy(buf0.at[pl.ds(0, rem)],
                            acc.at[pl.ds(s * RPT + (RPT // SUB) * SUB, rem)])

        plsc.subcore_barrier()

        def gather(j, buf, sem):
            return pltpu.make_async_copy(feat_hbm.at[src_v.at[j]], buf, sem)

        def scale(buf, j):
            def gbody(g, carry):
                w16 = w_v[j, pl.ds(g * LANES, LANES)]
                for i in range(LANES):
                    wgt = w16[i]
                    e = g * LANES + i
                    for d in range(DIN // LANES):
                        sl = pl.ds(d * LANES, LANES)
                        buf[e, sl] = buf[e, sl] * wgt
                return carry
            lax.fori_loop(0, SUB // LANES, gbody, 0)

        def scatter(buf, j):
            pltpu.sync_copy(buf, acc.at[dst_v.at[j]], add=True)

        for ph in range(PHASES):
            pbase = wid * NSUB + ph * PNS
            pltpu.sync_copy(src_hbm.at[pl.ds(pbase, PNS)], src_v)
            pltpu.sync_copy(dst_hbm.at[pl.ds(pbase, PNS)], dst_v)
            pltpu.sync_copy(w_hbm.at[pl.ds(pbase, PNS)], w_v)

            gather(0, buf0, sem0).start()

            def pbody(p, carry):
                j0 = 2 * p
                j1 = j0 + 1
                gather(j1, buf1, sem1).start()
                gather(j0, buf0, sem0).wait()
                scale(buf0, j0)
                scatter(buf0, j0)

                @pl.when(p < PPAIR - 1)
                def _():
                    gather(j0 + 2, buf0, sem0).start()

                gather(j1, buf1, sem1).wait()
                scale(buf1, j1)
                scatter(buf1, j1)
                return carry

            lax.fori_loop(0, PPAIR, pbody, 0)

        plsc.subcore_barrier()
        pltpu.sync_copy(acc.at[pl.ds(s * RPT, RPT)],
                        out_hbm.at[c, pl.ds(s * RPT, RPT)])

    return sc_spmm


def _epilogue(partials, W, N, DIN, DOUT):
    BLK = 1000

    def body(p_ref, w_ref, o_ref):
        x = p_ref[0] + p_ref[1]
        y = jnp.dot(x, w_ref[...], preferred_element_type=jnp.float32)
        o_ref[...] = jnp.maximum(y, 0.0)

    return pl.pallas_call(
        body,
        grid=(N // BLK,),
        in_specs=[
            pl.BlockSpec((2, BLK, DIN), lambda i: (0, i, 0)),
            pl.BlockSpec((DIN, DOUT), lambda i: (0, 0)),
        ],
        out_specs=pl.BlockSpec((BLK, DOUT), lambda i: (i, 0)),
        out_shape=jax.ShapeDtypeStruct((N, DOUT), jnp.float32),
    )(partials, W)


def kernel(features, edge_index, edge_weight, W):
    N, DIN = features.shape
    E = edge_index.shape[1]
    DOUT = W.shape[1]

    dst = edge_index[0]
    src = edge_index[1]

    group = 16 * NW * SUB
    EP = ((E + group - 1) // group) * group
    if EP != E:
        pad = EP - E
        src = jnp.concatenate([src, jnp.zeros((pad,), jnp.int32)])
        dst = jnp.concatenate([dst, jnp.zeros((pad,), jnp.int32)])
        edge_weight = jnp.concatenate(
            [edge_weight, jnp.zeros((pad,), jnp.float32)])

    src2d = src.reshape(EP // SUB, SUB)
    dst2d = dst.reshape(EP // SUB, SUB)
    w2d = edge_weight.reshape(EP // SUB, SUB)

    NP = ((N + NS * 8 - 1) // (NS * 8)) * (NS * 8)

    partials = _make_sc_spmm(NP, DIN, EP)(features, src2d, dst2d, w2d)
    return _epilogue(partials[:, :N, :], W, N, DIN, DOUT)


# speedup vs baseline: 3.7379x; 1.0614x over previous
import functools

import jax
import jax.numpy as jnp
from jax import lax
from jax.experimental import pallas as pl
from jax.experimental.pallas import tpu as pltpu
from jax.experimental.pallas import tpu_sc as plsc

NC = 2
NS = 16
NW = NC * NS
LANES = 16
SUB = 128
PNS = 40
BIG_CORE = 0
BIG_QUARTERS = 3


def _make_sc_spmm(NP, DIN, EP):
    TSUB = EP // SUB
    SPT = TSUB // NS
    NSUB_BIG = (SPT * BIG_QUARTERS // 4) // PNS * PNS
    NSUB_SMALL = SPT - NSUB_BIG
    assert NSUB_SMALL % PNS == 0 and NSUB_BIG % PNS == 0
    MAXPH = NSUB_BIG // PNS
    PPAIR = PNS // 2
    RPT = NP // NS

    mesh = plsc.VectorSubcoreMesh(core_axis_name="c", subcore_axis_name="s")

    @functools.partial(
        pl.kernel,
        out_type=jax.ShapeDtypeStruct((NC, NP, DIN), jnp.float32),
        mesh=mesh,
        scratch_types=[
            pltpu.VMEM_SHARED((NP, DIN), jnp.float32),
            pltpu.VMEM((PNS, SUB), jnp.int32),
            pltpu.VMEM((PNS, SUB), jnp.int32),
            pltpu.VMEM((PNS, SUB), jnp.float32),
            pltpu.VMEM((SUB, DIN), jnp.float32),
            pltpu.VMEM((SUB, DIN), jnp.float32),
            pltpu.SemaphoreType.DMA,
            pltpu.SemaphoreType.DMA,
        ],
    )
    def sc_spmm(feat_hbm, src_hbm, dst_hbm, w_hbm, out_hbm,
                acc, src_v, dst_v, w_v, buf0, buf1, sem0, sem1):
        c = lax.axis_index("c")
        s = lax.axis_index("s")
        is_big = c == BIG_CORE
        nsub = jnp.where(is_big, NSUB_BIG, NSUB_SMALL)
        nph = nsub // PNS
        tile_base = jnp.where(is_big, s * NSUB_BIG,
                              NS * NSUB_BIG + s * NSUB_SMALL)

        zero16 = jnp.zeros((LANES,), jnp.float32)

        def zbody(i, carry):
            for d in range(DIN // LANES):
                buf0[i, pl.ds(d * LANES, LANES)] = zero16
            return carry

        lax.fori_loop(0, SUB, zbody, 0)
        for z in range(RPT // SUB):
            pltpu.sync_copy(buf0, acc.at[pl.ds(s * RPT + z * SUB, SUB)])
        rem = RPT % SUB
        if rem:
            pltpu.sync_copy(buf0.at[pl.ds(0, rem)],
                            acc.at[pl.ds(s * RPT + (RPT // SUB) * SUB, rem)])

        plsc.subcore_barrier()

        def gather(j, buf, sem):
            return pltpu.make_async_copy(feat_hbm.at[src_v.at[j]], buf, sem)

        def scale(buf, j):
            def gbody(g, carry):
                w16 = w_v[j, pl.ds(g * LANES, LANES)]
                for i in range(LANES):
                    wgt = w16[i]
                    e = g * LANES + i
                    for d in range(DIN // LANES):
                        sl = pl.ds(d * LANES, LANES)
                        buf[e, sl] = buf[e, sl] * wgt
                return carry
            lax.fori_loop(0, SUB // LANES, gbody, 0)

        def scatter(buf, j):
            pltpu.sync_copy(buf, acc.at[dst_v.at[j]], add=True)

        def pbody(p, carry):
            j0 = 2 * p
            j1 = j0 + 1
            gather(j1, buf1, sem1).start()
            gather(j0, buf0, sem0).wait()
            scale(buf0, j0)
            scatter(buf0, j0)

            @pl.when(p < PPAIR - 1)
            def _():
                gather(j0 + 2, buf0, sem0).start()

            gather(j1, buf1, sem1).wait()
            scale(buf1, j1)
            scatter(buf1, j1)
            return carry

        for ph in range(MAXPH):
            @pl.when(ph < nph)
            def _run_phase():
                pbase = pl.multiple_of(tile_base + ph * PNS, 8)
                pltpu.sync_copy(src_hbm.at[pl.ds(pbase, PNS)], src_v)
                pltpu.sync_copy(dst_hbm.at[pl.ds(pbase, PNS)], dst_v)
                pltpu.sync_copy(w_hbm.at[pl.ds(pbase, PNS)], w_v)

                gather(0, buf0, sem0).start()
                lax.fori_loop(0, PPAIR, pbody, 0)

        plsc.subcore_barrier()
        pltpu.sync_copy(acc.at[pl.ds(s * RPT, RPT)],
                        out_hbm.at[c, pl.ds(s * RPT, RPT)])

    return sc_spmm


def _epilogue(partials, W, N, DIN, DOUT):
    BLK = 1000

    def body(p_ref, w_ref, o_ref):
        x = p_ref[0] + p_ref[1]
        y = jnp.dot(x, w_ref[...], preferred_element_type=jnp.float32)
        o_ref[...] = jnp.maximum(y, 0.0)

    return pl.pallas_call(
        body,
        grid=(N // BLK,),
        in_specs=[
            pl.BlockSpec((2, BLK, DIN), lambda i: (0, i, 0)),
            pl.BlockSpec((DIN, DOUT), lambda i: (0, 0)),
        ],
        out_specs=pl.BlockSpec((BLK, DOUT), lambda i: (i, 0)),
        out_shape=jax.ShapeDtypeStruct((N, DOUT), jnp.float32),
    )(partials, W)


def kernel(features, edge_index, edge_weight, W):
    N, DIN = features.shape
    E = edge_index.shape[1]
    DOUT = W.shape[1]

    dst = edge_index[0]
    src = edge_index[1]

    group = SUB * NS * PNS
    EP = ((E + group - 1) // group) * group
    if EP != E:
        pad = EP - E
        src = jnp.concatenate([src, jnp.zeros((pad,), jnp.int32)])
        dst = jnp.concatenate([dst, jnp.zeros((pad,), jnp.int32)])
        edge_weight = jnp.concatenate(
            [edge_weight, jnp.zeros((pad,), jnp.float32)])

    src2d = src.reshape(EP // SUB, SUB)
    dst2d = dst.reshape(EP // SUB, SUB)
    w2d = edge_weight.reshape(EP // SUB, SUB)

    NP = ((N + NS * 8 - 1) // (NS * 8)) * (NS * 8)

    partials = _make_sc_spmm(NP, DIN, EP)(features, src2d, dst2d, w2d)
    return _epilogue(partials[:, :N, :], W, N, DIN, DOUT)
